# Initial kernel scaffold; baseline (speedup 1.0000x reference)
import jax
import jax.numpy as jnp
from jax.experimental import pallas as pl


def kernel(x, edge_index, W_mu, b_mu, W_logstd, b_logstd):
    raise NotImplementedError("write your pallas kernel here")


# speedup vs baseline: 14.1518x; 14.1518x over previous
import functools

import jax
import jax.numpy as jnp
from jax import lax
from jax.experimental import pallas as pl
from jax.experimental.pallas import tpu as pltpu
from jax.experimental.pallas import tpu_sc as plsc

N_NODES = 10000
D_IN = 128
D_OUT = 128
N_EDGES = 320000

NW = 32
N_PAD = 10240
ROWS_SUB = N_PAD // 16
E_PAD = NW * 10240
PER_W = E_PAD // NW
CHUNK = 128
CHUNKS = PER_W // CHUNK
INIT_REPS = ROWS_SUB // CHUNK

_MESH = plsc.VectorSubcoreMesh(core_axis_name="c", subcore_axis_name="s")


def _zero16():
    return jnp.zeros((16,), jnp.float32)


def _deg_kernel(dst_hbm, out_hbm, shared, buf, idx):
    c = lax.axis_index("c")
    s = lax.axis_index("s")
    wid = s * 2 + c

    def fill_zero(i, _):
        buf[i, :] = _zero16()
        return 0

    lax.fori_loop(0, CHUNK, fill_zero, 0)
    for r in range(INIT_REPS):
        pltpu.sync_copy(buf, shared.at[pl.ds(s * ROWS_SUB + r * CHUNK, CHUNK)])

    def fill_one(i, _):
        buf[i, :] = _zero16() + 1.0
        return 0

    lax.fori_loop(0, CHUNK, fill_one, 0)
    plsc.subcore_barrier()

    base = wid * PER_W

    def body(g, _):
        pltpu.sync_copy(dst_hbm.at[pl.ds(base + g * CHUNK, CHUNK)], idx)
        pltpu.sync_copy(buf, shared.at[idx], add=True)
        return 0

    lax.fori_loop(0, CHUNKS, body, 0)
    plsc.subcore_barrier()
    pltpu.sync_copy(
        shared.at[pl.ds(s * ROWS_SUB, ROWS_SUB)],
        out_hbm.at[pl.ds(c * N_PAD + s * ROWS_SUB, ROWS_SUB)],
    )


_deg_call = functools.partial(
    pl.kernel,
    mesh=_MESH,
    out_type=jax.ShapeDtypeStruct((2 * N_PAD, 16), jnp.float32),
    scratch_types=[
        pltpu.VMEM_SHARED((N_PAD, 16), jnp.float32),
        pltpu.VMEM((CHUNK, 16), jnp.float32),
        pltpu.VMEM((CHUNK,), jnp.int32),
    ],
)(_deg_kernel)


def _agg_kernel(z_hbm, src_hbm, dst_hbm, out_hbm, shared, rows, sidx, didx):
    c = lax.axis_index("c")
    s = lax.axis_index("s")
    wid = s * 2 + c

    def fill_zero(i, _):
        for d in range(8):
            rows[i, pl.ds(d * 16, 16)] = _zero16()
        return 0

    lax.fori_loop(0, CHUNK, fill_zero, 0)
    for r in range(INIT_REPS):
        pltpu.sync_copy(rows, shared.at[pl.ds(s * ROWS_SUB + r * CHUNK, CHUNK)])
    plsc.subcore_barrier()

    base = wid * PER_W

    def body(g, _):
        pltpu.sync_copy(src_hbm.at[pl.ds(base + g * CHUNK, CHUNK)], sidx)
        pltpu.sync_copy(dst_hbm.at[pl.ds(base + g * CHUNK, CHUNK)], didx)
        pltpu.sync_copy(z_hbm.at[sidx], rows)
        pltpu.sync_copy(rows, shared.at[didx], add=True)
        return 0

    lax.fori_loop(0, CHUNKS, body, 0)
    plsc.subcore_barrier()
    pltpu.sync_copy(
        shared.at[pl.ds(s * ROWS_SUB, ROWS_SUB)],
        out_hbm.at[pl.ds(c * N_PAD + s * ROWS_SUB, ROWS_SUB)],
    )


_agg_call = functools.partial(
    pl.kernel,
    mesh=_MESH,
    out_type=jax.ShapeDtypeStruct((2 * N_PAD, D_IN), jnp.float32),
    scratch_types=[
        pltpu.VMEM_SHARED((N_PAD, D_IN), jnp.float32),
        pltpu.VMEM((CHUNK, D_IN), jnp.float32),
        pltpu.VMEM((CHUNK,), jnp.int32),
        pltpu.VMEM((CHUNK,), jnp.int32),
    ],
)(_agg_kernel)


def _scale_kernel(degp_ref, x_ref, z_ref, deg_ref):
    deg = (
        degp_ref[0:N_NODES, 0:1]
        + degp_ref[N_PAD : N_PAD + N_NODES, 0:1]
        + 1.0
    )
    z_ref[...] = x_ref[...] * lax.rsqrt(deg)
    deg_ref[...] = deg


def _scale_call(degp, x):
    return pl.pallas_call(
        _scale_kernel,
        out_shape=[
            jax.ShapeDtypeStruct((N_NODES, D_IN), jnp.float32),
            jax.ShapeDtypeStruct((N_NODES, 1), jnp.float32),
        ],
    )(degp, x)


_BLK = 1024
_GRID = (N_NODES + _BLK - 1) // _BLK


def _out_kernel(p0_ref, p1_ref, x_ref, deg_ref, w_ref, b_ref, out_ref):
    deg = deg_ref[...]
    p = p0_ref[...] + p1_ref[...]
    yy = p * lax.rsqrt(deg) + x_ref[...] * (1.0 / deg)
    out_ref[...] = (
        jnp.dot(yy, w_ref[...], preferred_element_type=jnp.float32) + b_ref[...]
    )


def _out_call(p, x, deg, wcat, bcat):
    return pl.pallas_call(
        _out_kernel,
        grid=(_GRID,),
        in_specs=[
            pl.BlockSpec((_BLK, D_IN), lambda i: (i, 0)),
            pl.BlockSpec((_BLK, D_IN), lambda i: (i + N_PAD // _BLK, 0)),
            pl.BlockSpec((_BLK, D_IN), lambda i: (i, 0)),
            pl.BlockSpec((_BLK, 1), lambda i: (i, 0)),
            pl.BlockSpec((D_IN, 2 * D_OUT), lambda i: (0, 0)),
            pl.BlockSpec((1, 2 * D_OUT), lambda i: (0, 0)),
        ],
        out_specs=pl.BlockSpec((_BLK, 2 * D_OUT), lambda i: (i, 0)),
        out_shape=jax.ShapeDtypeStruct((N_NODES, 2 * D_OUT), jnp.float32),
    )(p, p, x, deg, wcat, bcat)


def kernel(x, edge_index, W_mu, b_mu, W_logstd, b_logstd):
    ei = edge_index.astype(jnp.int32)
    pad = E_PAD - N_EDGES
    src = jnp.concatenate([ei[0], jnp.zeros((pad,), jnp.int32)])
    dst = jnp.concatenate([ei[1], jnp.full((pad,), N_NODES, jnp.int32)])

    degp = _deg_call(dst)
    z, deg = _scale_call(degp, x)
    p = _agg_call(z, src, dst)

    wcat = jnp.concatenate([W_mu, W_logstd], axis=1)
    bcat = jnp.concatenate([b_mu, b_logstd]).reshape(1, 2 * D_OUT)
    out = _out_call(p, x, deg, wcat, bcat)
    return (out[:, :D_OUT], out[:, D_OUT:])
